# baseline (speedup 1.0000x reference)
import jax
import jax.numpy as jnp
from jax import lax
from jax.experimental import pallas as pl
from jax.experimental.pallas import tpu as pltpu
from jax.experimental.pallas import tpu_sc as plsc

_NN = 8
_NF = 16
_NT_PAD = 128
_D = 3 + 3 * _NF
_NPAIR = _NN * (_NN - 1)
_BC = 2048

_NA = 100000
_C = 20000
_NC = 2
_NS = 16
_NW = _NC * _NS
_CNT0 = 624
_CNT_LAST = _C - _CNT0 * (_NW - 1)
_CNT_MAX = _CNT_LAST


_STG = 6256


def _sc_gather_body(xt_hbm, yt_hbm, zt_hbm, tidf_hbm, jidx_hbm, iidx_hbm,
                    dist_hbm,
                    xs_out, ys_out, zs_out, tidj_out, tidi_out, dist_out,
                    table_v, jidx_v, iidx_v, outj_v, outi_v, chunk_v):
    wid = lax.axis_index("s") * _NC + lax.axis_index("c")

    def gather_chunks(idx_ref, out_ref, n16):
        def body(i, carry):
            s = pl.multiple_of(i * 16, 16)
            idx = idx_ref[pl.ds(s, 16)]
            out_ref[pl.ds(s, 16)] = plsc.load_gather(table_v, [idx])
            return carry
        lax.fori_loop(0, n16, body, 0, unroll=4)

    def run(base, cnt):
        pltpu.sync_copy(dist_hbm.at[pl.ds(base * _NN, cnt * _NN)],
                        chunk_v.at[pl.ds(0, cnt * _NN)])
        lane = lax.iota(jnp.int32, 16)
        pat = (lane % _NN) * cnt + lane // _NN

        def tr_body(i, carry):
            s = pl.multiple_of(i * 16, 16)
            vals = chunk_v[pl.ds(s, 16)]
            plsc.store_scatter(outj_v, [pat + 2 * i], vals)
            return carry
        lax.fori_loop(0, cnt * _NN // 16, tr_body, 0, unroll=4)
        for m in range(_NN):
            dst = pl.multiple_of(m * _C + base, 8)
            pltpu.sync_copy(outj_v.at[pl.ds(m * cnt, cnt)],
                            dist_out.at[pl.ds(dst, cnt)])

        for m in range(_NN):
            src = pl.multiple_of(m * _C + base, 8)
            pltpu.sync_copy(jidx_hbm.at[pl.ds(src, cnt)],
                            jidx_v.at[pl.ds(m * cnt, cnt)])
        pltpu.sync_copy(iidx_hbm.at[pl.ds(base, cnt)], iidx_v.at[pl.ds(0, cnt)])
        for table_hbm, out_hbm, with_i in (
                (xt_hbm, xs_out, False), (yt_hbm, ys_out, False),
                (zt_hbm, zs_out, False), (tidf_hbm, tidj_out, True)):
            pltpu.sync_copy(table_hbm, table_v)
            gather_chunks(jidx_v, outj_v, cnt * _NN // 16)
            for m in range(_NN):
                dst = pl.multiple_of(m * _C + base, 8)
                pltpu.sync_copy(outj_v.at[pl.ds(m * cnt, cnt)],
                                out_hbm.at[pl.ds(dst, cnt)])
            if with_i:
                gather_chunks(iidx_v, outi_v, cnt // 16)
                pltpu.sync_copy(outi_v.at[pl.ds(0, cnt)],
                                tidi_out.at[pl.ds(base, cnt)])

    @pl.when(wid < _NW - 1)
    def _():
        run(pl.multiple_of(wid * _CNT0, 16), _CNT0)

    @pl.when(wid == _NW - 1)
    def _():
        run((_NW - 1) * _CNT0, _CNT_LAST)


def _sc_gather(atoms_xyz_t, tid_f32, jidx_t_flat, atom_i_idx, dist_flat):
    f32 = jnp.float32
    mesh = plsc.VectorSubcoreMesh(core_axis_name="c", subcore_axis_name="s")
    nj_max = _CNT_MAX * _NN
    call = pl.kernel(
        _sc_gather_body,
        out_type=(
            jax.ShapeDtypeStruct((_NN * _C,), f32),
            jax.ShapeDtypeStruct((_NN * _C,), f32),
            jax.ShapeDtypeStruct((_NN * _C,), f32),
            jax.ShapeDtypeStruct((_NN * _C,), f32),
            jax.ShapeDtypeStruct((_C,), f32),
            jax.ShapeDtypeStruct((_NN * _C,), f32),
        ),
        mesh=mesh,
        scratch_types=[
            pltpu.VMEM((_NA,), f32),
            pltpu.VMEM((nj_max,), jnp.int32),
            pltpu.VMEM((_CNT_MAX,), jnp.int32),
            pltpu.VMEM((nj_max,), f32),
            pltpu.VMEM((_CNT_MAX,), f32),
            pltpu.VMEM((nj_max,), f32),
        ],
        compiler_params=pltpu.CompilerParams(needs_layout_passes=False),
    )
    return call(atoms_xyz_t[0], atoms_xyz_t[1], atoms_xyz_t[2],
                tid_f32, jidx_t_flat, atom_i_idx, dist_flat)


def _dense_body(dist_ref, xs_ref, ys_ref, zs_ref, tidi_ref, tidj_ref,
                emb_ref, out_ref):
    Bc = dist_ref.shape[1]
    dist = dist_ref[...]
    xs = xs_ref[...]
    ys = ys_ref[...]
    zs = zs_ref[...]
    embT = emb_ref[...]

    sub_iota = jax.lax.broadcasted_iota(jnp.int32, (_NT_PAD, Bc), 0)

    def emb_lookup(tid_row):
        oh = (sub_iota == tid_row).astype(jnp.float32)
        return jax.lax.dot_general(
            embT, oh, (((1,), (0,)), ((), ())),
            precision=jax.lax.Precision.HIGHEST,
            preferred_element_type=jnp.float32)

    e_i = emb_lookup(tidi_ref[...])
    ep = [emb_lookup(tidj_ref[m:m + 1, :]) / dist[m:m + 1, :]
          for m in range(_NN)]

    tjk = []
    for j in range(_NN):
        dx = xs - xs[j:j + 1, :]
        dy = ys - ys[j:j + 1, :]
        dz = zs - zs[j:j + 1, :]
        raw = jnp.sqrt(dx * dx + dy * dy + dz * dz + 1e-12)
        tij = dist[j:j + 1, :]
        mx = jnp.maximum(tij, dist)
        mn = jnp.minimum(tij, dist)
        tjk.append((raw - mx + mn) / (2.0 * mn))

    def rep7(row):
        return jnp.broadcast_to(row, (7, Bc))

    planes = []
    planes.append(jnp.concatenate([rep7(dist[j:j + 1, :])
                                   for j in range(_NN)], axis=0))
    planes.append(jnp.concatenate([dist[k:k + 1, :]
                                   for j in range(_NN)
                                   for k in range(_NN) if k != j], axis=0))
    planes.append(jnp.concatenate([tjk[j][k:k + 1, :]
                                   for j in range(_NN)
                                   for k in range(_NN) if k != j], axis=0))
    for f in range(_NF):
        planes.append(jnp.broadcast_to(e_i[f:f + 1, :], (_NPAIR, Bc)))
    for f in range(_NF):
        planes.append(jnp.concatenate([rep7(ep[j][f:f + 1, :])
                                       for j in range(_NN)], axis=0))
    for f in range(_NF):
        planes.append(jnp.concatenate([ep[k][f:f + 1, :]
                                       for j in range(_NN)
                                       for k in range(_NN) if k != j], axis=0))
    for f in range(_D):
        out_ref[f] = planes[f]


def _dense_call(dist_t, xs_t, ys_t, zs_t, tid_i_row, tid_j_t, emb_t_pad):
    grid = (pl.cdiv(_C, _BC),)
    row_spec = pl.BlockSpec((_NN, _BC), lambda i: (0, i))
    out = pl.pallas_call(
        _dense_body,
        grid=grid,
        in_specs=[
            row_spec,
            row_spec, row_spec, row_spec,
            pl.BlockSpec((1, _BC), lambda i: (0, i)),
            row_spec,
            pl.BlockSpec((_NF, _NT_PAD), lambda i: (0, 0)),
        ],
        out_specs=pl.BlockSpec((_D, _NPAIR, _BC), lambda i: (0, 0, i)),
        out_shape=jax.ShapeDtypeStruct((_D, _NPAIR, _C), jnp.float32),
    )(dist_t, xs_t, ys_t, zs_t, tid_i_row, tid_j_t, emb_t_pad)
    return out


def kernel(atoms_xyz, embed_table, dist_ij, atom_type_ids, atom_i_idx, atom_j_idx):
    C, n = dist_ij.shape
    atoms_xyz_t = atoms_xyz.T.reshape(3, -1)
    tid_f32 = lax.bitcast_convert_type(atom_type_ids.astype(jnp.int32),
                                       jnp.float32)
    jidx_t_flat = atom_j_idx.astype(jnp.int32).T.reshape(-1)
    xs, ys, zs, tidj_f, tidi_f, dist_f = _sc_gather(
        atoms_xyz_t, tid_f32, jidx_t_flat, atom_i_idx.astype(jnp.int32),
        dist_ij.reshape(-1))
    xs_t = xs.reshape(n, C)
    ys_t = ys.reshape(n, C)
    zs_t = zs.reshape(n, C)
    tid_j_t = lax.bitcast_convert_type(tidj_f, jnp.int32).reshape(n, C)
    tid_i_row = lax.bitcast_convert_type(tidi_f, jnp.int32).reshape(1, C)
    emb_t_pad = jnp.zeros((_NF, _NT_PAD), jnp.float32).at[:, :embed_table.shape[0]].set(embed_table.T)

    out_t = _dense_call(dist_f.reshape(n, C), xs_t, ys_t, zs_t, tid_i_row,
                        tid_j_t, emb_t_pad)
    ang = out_t.transpose(2, 1, 0)
    return (ang, atom_i_idx.reshape(-1))


# speedup vs baseline: 1.1116x; 1.1116x over previous
import jax
import jax.numpy as jnp
from jax import lax
from jax.experimental import pallas as pl
from jax.experimental.pallas import tpu as pltpu
from jax.experimental.pallas import tpu_sc as plsc

_NN = 8
_NF = 16
_NT_PAD = 128
_D = 3 + 3 * _NF
_NPAIR = _NN * (_NN - 1)
_BC = 2048

_NA = 100000
_C = 20000
_NC = 2
_NS = 16
_NW = _NC * _NS
_CNT0 = 624
_CNT_LAST = _C - _CNT0 * (_NW - 1)
_CNT_MAX = _CNT_LAST


def _sc_gather_body(xt_hbm, yt_hbm, zt_hbm, tidf_hbm, jidx_hbm, iidx_hbm,
                    xs_out, ys_out, zs_out, tidj_out, tidi_out,
                    table_v, jidx_v, iidx_v, outj_v, outi_v):
    wid = lax.axis_index("s") * _NC + lax.axis_index("c")

    def gather_chunks(idx_ref, out_ref, n16):
        def body(i, carry):
            s = pl.multiple_of(i * 16, 16)
            idx = idx_ref[pl.ds(s, 16)]
            out_ref[pl.ds(s, 16)] = plsc.load_gather(table_v, [idx])
            return carry
        lax.fori_loop(0, n16, body, 0, unroll=4)

    def run(base, cnt):
        for m in range(_NN):
            src = pl.multiple_of(m * _C + base, 8)
            pltpu.sync_copy(jidx_hbm.at[pl.ds(src, cnt)],
                            jidx_v.at[pl.ds(m * cnt, cnt)])
        pltpu.sync_copy(iidx_hbm.at[pl.ds(base, cnt)], iidx_v.at[pl.ds(0, cnt)])
        for table_hbm, out_hbm, with_i in (
                (xt_hbm, xs_out, False), (yt_hbm, ys_out, False),
                (zt_hbm, zs_out, False), (tidf_hbm, tidj_out, True)):
            pltpu.sync_copy(table_hbm, table_v)
            gather_chunks(jidx_v, outj_v, cnt * _NN // 16)
            for m in range(_NN):
                dst = pl.multiple_of(m * _C + base, 8)
                pltpu.sync_copy(outj_v.at[pl.ds(m * cnt, cnt)],
                                out_hbm.at[pl.ds(dst, cnt)])
            if with_i:
                gather_chunks(iidx_v, outi_v, cnt // 16)
                pltpu.sync_copy(outi_v.at[pl.ds(0, cnt)],
                                tidi_out.at[pl.ds(base, cnt)])

    @pl.when(wid < _NW - 1)
    def _():
        run(pl.multiple_of(wid * _CNT0, 16), _CNT0)

    @pl.when(wid == _NW - 1)
    def _():
        run((_NW - 1) * _CNT0, _CNT_LAST)


def _sc_gather(atoms_xyz_t, tid_f32, jidx_t_flat, atom_i_idx):
    f32 = jnp.float32
    mesh = plsc.VectorSubcoreMesh(core_axis_name="c", subcore_axis_name="s")
    nj_max = _CNT_MAX * _NN
    call = pl.kernel(
        _sc_gather_body,
        out_type=(
            jax.ShapeDtypeStruct((_NN * _C,), f32),
            jax.ShapeDtypeStruct((_NN * _C,), f32),
            jax.ShapeDtypeStruct((_NN * _C,), f32),
            jax.ShapeDtypeStruct((_NN * _C,), f32),
            jax.ShapeDtypeStruct((_C,), f32),
        ),
        mesh=mesh,
        scratch_types=[
            pltpu.VMEM((_NA,), f32),
            pltpu.VMEM((nj_max,), jnp.int32),
            pltpu.VMEM((_CNT_MAX,), jnp.int32),
            pltpu.VMEM((nj_max,), f32),
            pltpu.VMEM((_CNT_MAX,), f32),
        ],
        compiler_params=pltpu.CompilerParams(needs_layout_passes=False),
    )
    return call(atoms_xyz_t[0], atoms_xyz_t[1], atoms_xyz_t[2],
                tid_f32, jidx_t_flat, atom_i_idx)


def _dense_body(dist_ref, xs_ref, ys_ref, zs_ref, tidi_ref, tidj_ref,
                emb_ref, out_ref):
    Bc = dist_ref.shape[1]
    dist = dist_ref[...]
    xs = xs_ref[...]
    ys = ys_ref[...]
    zs = zs_ref[...]
    embT = emb_ref[...]

    sub_iota = jax.lax.broadcasted_iota(jnp.int32, (_NT_PAD, Bc), 0)

    def emb_lookup(tid_row):
        oh = (sub_iota == tid_row).astype(jnp.float32)
        return jax.lax.dot_general(
            embT, oh, (((1,), (0,)), ((), ())),
            precision=jax.lax.Precision.HIGHEST,
            preferred_element_type=jnp.float32)

    e_i = emb_lookup(tidi_ref[...])
    ep = [emb_lookup(tidj_ref[m:m + 1, :]) / dist[m:m + 1, :]
          for m in range(_NN)]

    tjk = []
    for j in range(_NN):
        dx = xs - xs[j:j + 1, :]
        dy = ys - ys[j:j + 1, :]
        dz = zs - zs[j:j + 1, :]
        raw = jnp.sqrt(dx * dx + dy * dy + dz * dz + 1e-12)
        tij = dist[j:j + 1, :]
        mx = jnp.maximum(tij, dist)
        mn = jnp.minimum(tij, dist)
        tjk.append((raw - mx + mn) / (2.0 * mn))

    def rep7(row):
        return jnp.broadcast_to(row, (7, Bc))

    planes = []
    planes.append(jnp.concatenate([rep7(dist[j:j + 1, :])
                                   for j in range(_NN)], axis=0))
    planes.append(jnp.concatenate([dist[k:k + 1, :]
                                   for j in range(_NN)
                                   for k in range(_NN) if k != j], axis=0))
    planes.append(jnp.concatenate([tjk[j][k:k + 1, :]
                                   for j in range(_NN)
                                   for k in range(_NN) if k != j], axis=0))
    for f in range(_NF):
        planes.append(jnp.broadcast_to(e_i[f:f + 1, :], (_NPAIR, Bc)))
    for f in range(_NF):
        planes.append(jnp.concatenate([rep7(ep[j][f:f + 1, :])
                                       for j in range(_NN)], axis=0))
    for f in range(_NF):
        planes.append(jnp.concatenate([ep[k][f:f + 1, :]
                                       for j in range(_NN)
                                       for k in range(_NN) if k != j], axis=0))
    for f in range(_D):
        out_ref[f] = planes[f]


def _dense_call(dist_t, xs_t, ys_t, zs_t, tid_i_row, tid_j_t, emb_t_pad):
    grid = (pl.cdiv(_C, _BC),)
    row_spec = pl.BlockSpec((_NN, _BC), lambda i: (0, i))
    out = pl.pallas_call(
        _dense_body,
        grid=grid,
        in_specs=[
            row_spec,
            row_spec, row_spec, row_spec,
            pl.BlockSpec((1, _BC), lambda i: (0, i)),
            row_spec,
            pl.BlockSpec((_NF, _NT_PAD), lambda i: (0, 0)),
        ],
        out_specs=pl.BlockSpec((_D, _NPAIR, _BC), lambda i: (0, 0, i)),
        out_shape=jax.ShapeDtypeStruct((_D, _NPAIR, _C), jnp.float32),
    )(dist_t, xs_t, ys_t, zs_t, tid_i_row, tid_j_t, emb_t_pad)
    return out


def kernel(atoms_xyz, embed_table, dist_ij, atom_type_ids, atom_i_idx, atom_j_idx):
    C, n = dist_ij.shape
    atoms_xyz_t = atoms_xyz.T.reshape(3, -1)
    tid_f32 = lax.bitcast_convert_type(atom_type_ids.astype(jnp.int32),
                                       jnp.float32)
    jidx_t_flat = atom_j_idx.astype(jnp.int32).T.reshape(-1)
    xs, ys, zs, tidj_f, tidi_f = _sc_gather(
        atoms_xyz_t, tid_f32, jidx_t_flat, atom_i_idx.astype(jnp.int32))
    xs_t = xs.reshape(n, C)
    ys_t = ys.reshape(n, C)
    zs_t = zs.reshape(n, C)
    tid_j_t = lax.bitcast_convert_type(tidj_f, jnp.int32).reshape(n, C)
    tid_i_row = lax.bitcast_convert_type(tidi_f, jnp.int32).reshape(1, C)
    emb_t_pad = jnp.zeros((_NF, _NT_PAD), jnp.float32).at[:, :embed_table.shape[0]].set(embed_table.T)

    out_t = _dense_call(dist_ij.T, xs_t, ys_t, zs_t, tid_i_row,
                        tid_j_t, emb_t_pad)
    ang = out_t.transpose(2, 1, 0)
    return (ang, atom_i_idx.reshape(-1))


# speedup vs baseline: 1.1702x; 1.0527x over previous
import jax
import jax.numpy as jnp
from jax import lax
from jax.experimental import pallas as pl
from jax.experimental.pallas import tpu as pltpu
from jax.experimental.pallas import tpu_sc as plsc

_NN = 8
_NF = 16
_NT_PAD = 128
_D = 3 + 3 * _NF
_NPAIR = _NN * (_NN - 1)
_BC = 2048

_NA = 100000
_C = 20000
_NC = 2
_NS = 16
_NW = _NC * _NS
_CNT0 = 624
_CNT_LAST = _C - _CNT0 * (_NW - 1)
_CNT_MAX = _CNT_LAST


def _sc_gather_body(xt_hbm, yt_hbm, zt_hbm, tidf_hbm, jidx_hbm, iidx_hbm,
                    xs_out, ys_out, zs_out, tidj_out, tidi_out,
                    table_v, jidx_v, iidx_v, outj_v, outi_v):
    wid = lax.axis_index("s") * _NC + lax.axis_index("c")

    def gather_chunks(idx_ref, out_ref, n16):
        @plsc.parallel_loop(0, n16 * 16, 16, unroll=8)
        def _(s):
            idx = idx_ref[pl.ds(s, 16)]
            out_ref[pl.ds(s, 16)] = plsc.load_gather(table_v, [idx])

    def run(base, cnt):
        for m in range(_NN):
            src = pl.multiple_of(m * _C + base, 8)
            pltpu.sync_copy(jidx_hbm.at[pl.ds(src, cnt)],
                            jidx_v.at[pl.ds(m * cnt, cnt)])
        pltpu.sync_copy(iidx_hbm.at[pl.ds(base, cnt)], iidx_v.at[pl.ds(0, cnt)])
        for table_hbm, out_hbm, with_i in (
                (xt_hbm, xs_out, False), (yt_hbm, ys_out, False),
                (zt_hbm, zs_out, False), (tidf_hbm, tidj_out, True)):
            pltpu.sync_copy(table_hbm, table_v)
            gather_chunks(jidx_v, outj_v, cnt * _NN // 16)
            for m in range(_NN):
                dst = pl.multiple_of(m * _C + base, 8)
                pltpu.sync_copy(outj_v.at[pl.ds(m * cnt, cnt)],
                                out_hbm.at[pl.ds(dst, cnt)])
            if with_i:
                gather_chunks(iidx_v, outi_v, cnt // 16)
                pltpu.sync_copy(outi_v.at[pl.ds(0, cnt)],
                                tidi_out.at[pl.ds(base, cnt)])

    @pl.when(wid < _NW - 1)
    def _():
        run(pl.multiple_of(wid * _CNT0, 16), _CNT0)

    @pl.when(wid == _NW - 1)
    def _():
        run((_NW - 1) * _CNT0, _CNT_LAST)


def _sc_gather(atoms_xyz_t, tid_f32, jidx_t_flat, atom_i_idx):
    f32 = jnp.float32
    mesh = plsc.VectorSubcoreMesh(core_axis_name="c", subcore_axis_name="s")
    nj_max = _CNT_MAX * _NN
    call = pl.kernel(
        _sc_gather_body,
        out_type=(
            jax.ShapeDtypeStruct((_NN * _C,), f32),
            jax.ShapeDtypeStruct((_NN * _C,), f32),
            jax.ShapeDtypeStruct((_NN * _C,), f32),
            jax.ShapeDtypeStruct((_NN * _C,), f32),
            jax.ShapeDtypeStruct((_C,), f32),
        ),
        mesh=mesh,
        scratch_types=[
            pltpu.VMEM((_NA,), f32),
            pltpu.VMEM((nj_max,), jnp.int32),
            pltpu.VMEM((_CNT_MAX,), jnp.int32),
            pltpu.VMEM((nj_max,), f32),
            pltpu.VMEM((_CNT_MAX,), f32),
        ],
        compiler_params=pltpu.CompilerParams(needs_layout_passes=False),
    )
    return call(atoms_xyz_t[0], atoms_xyz_t[1], atoms_xyz_t[2],
                tid_f32, jidx_t_flat, atom_i_idx)


def _dense_body(dist_ref, xs_ref, ys_ref, zs_ref, tidi_ref, tidj_ref,
                emb_ref, out_ref):
    Bc = dist_ref.shape[1]
    dist = dist_ref[...]
    xs = xs_ref[...]
    ys = ys_ref[...]
    zs = zs_ref[...]
    embT = emb_ref[...]

    sub_iota = jax.lax.broadcasted_iota(jnp.int32, (_NT_PAD, Bc), 0)

    def emb_lookup(tid_row):
        oh = (sub_iota == tid_row).astype(jnp.float32)
        return jax.lax.dot_general(
            embT, oh, (((1,), (0,)), ((), ())),
            precision=jax.lax.Precision.HIGHEST,
            preferred_element_type=jnp.float32)

    e_i = emb_lookup(tidi_ref[...])
    ep = [emb_lookup(tidj_ref[m:m + 1, :]) / dist[m:m + 1, :]
          for m in range(_NN)]

    tjk = []
    for j in range(_NN):
        dx = xs - xs[j:j + 1, :]
        dy = ys - ys[j:j + 1, :]
        dz = zs - zs[j:j + 1, :]
        raw = jnp.sqrt(dx * dx + dy * dy + dz * dz + 1e-12)
        tij = dist[j:j + 1, :]
        mx = jnp.maximum(tij, dist)
        mn = jnp.minimum(tij, dist)
        tjk.append((raw - mx + mn) / (2.0 * mn))

    def rep7(row):
        return jnp.broadcast_to(row, (7, Bc))

    planes = []
    planes.append(jnp.concatenate([rep7(dist[j:j + 1, :])
                                   for j in range(_NN)], axis=0))
    planes.append(jnp.concatenate([dist[k:k + 1, :]
                                   for j in range(_NN)
                                   for k in range(_NN) if k != j], axis=0))
    planes.append(jnp.concatenate([tjk[j][k:k + 1, :]
                                   for j in range(_NN)
                                   for k in range(_NN) if k != j], axis=0))
    for f in range(_NF):
        planes.append(jnp.broadcast_to(e_i[f:f + 1, :], (_NPAIR, Bc)))
    for f in range(_NF):
        planes.append(jnp.concatenate([rep7(ep[j][f:f + 1, :])
                                       for j in range(_NN)], axis=0))
    for f in range(_NF):
        planes.append(jnp.concatenate([ep[k][f:f + 1, :]
                                       for j in range(_NN)
                                       for k in range(_NN) if k != j], axis=0))
    for f in range(_D):
        out_ref[f] = planes[f]


def _dense_call(dist_t, xs_t, ys_t, zs_t, tid_i_row, tid_j_t, emb_t_pad):
    grid = (pl.cdiv(_C, _BC),)
    row_spec = pl.BlockSpec((_NN, _BC), lambda i: (0, i))
    out = pl.pallas_call(
        _dense_body,
        grid=grid,
        in_specs=[
            row_spec,
            row_spec, row_spec, row_spec,
            pl.BlockSpec((1, _BC), lambda i: (0, i)),
            row_spec,
            pl.BlockSpec((_NF, _NT_PAD), lambda i: (0, 0)),
        ],
        out_specs=pl.BlockSpec((_D, _NPAIR, _BC), lambda i: (0, 0, i)),
        out_shape=jax.ShapeDtypeStruct((_D, _NPAIR, _C), jnp.float32),
    )(dist_t, xs_t, ys_t, zs_t, tid_i_row, tid_j_t, emb_t_pad)
    return out


def kernel(atoms_xyz, embed_table, dist_ij, atom_type_ids, atom_i_idx, atom_j_idx):
    C, n = dist_ij.shape
    atoms_xyz_t = atoms_xyz.T.reshape(3, -1)
    tid_f32 = lax.bitcast_convert_type(atom_type_ids.astype(jnp.int32),
                                       jnp.float32)
    jidx_t_flat = atom_j_idx.astype(jnp.int32).T.reshape(-1)
    xs, ys, zs, tidj_f, tidi_f = _sc_gather(
        atoms_xyz_t, tid_f32, jidx_t_flat, atom_i_idx.astype(jnp.int32))
    xs_t = xs.reshape(n, C)
    ys_t = ys.reshape(n, C)
    zs_t = zs.reshape(n, C)
    tid_j_t = lax.bitcast_convert_type(tidj_f, jnp.int32).reshape(n, C)
    tid_i_row = lax.bitcast_convert_type(tidi_f, jnp.int32).reshape(1, C)
    emb_t_pad = jnp.zeros((_NF, _NT_PAD), jnp.float32).at[:, :embed_table.shape[0]].set(embed_table.T)

    out_t = _dense_call(dist_ij.T, xs_t, ys_t, zs_t, tid_i_row,
                        tid_j_t, emb_t_pad)
    ang = out_t.transpose(2, 1, 0)
    return (ang, atom_i_idx.reshape(-1))
